# baseline (speedup 1.0000x reference)
import functools

import jax
import jax.numpy as jnp
from jax import lax
from jax.experimental import pallas as pl
from jax.experimental.pallas import tpu as pltpu
from jax.experimental.pallas import tpu_sc as plsc

BATCH = 16384
CUST_VOCAB = 264055
CUST_DIM = 18
PROD_DIM = 7
DENSE_DIM = 13
CUST_PAD = 24
PROD_PAD = 8
WIN = 64
H0, H1 = 1024, 512

NC, NS = 2, 16
NW = NC * NS
B_PER_W = BATCH // NW
RCHUNK = 256

GRAN = 128
TAB_WORDS = CUST_VOCAB * CUST_DIM
TAB_GROWS = ((CUST_VOCAB - 1) * CUST_DIM) // GRAN + 2
TAB_PAD_WORDS = TAB_GROWS * GRAN - TAB_WORDS


def _sc_gather_body(cust_idx_hbm, prod_idx_hbm, w_cust_hbm, w_prod_hbm,
                    cust_out_hbm, prod_out_hbm,
                    cidx_v, pidx_v, glist_v, gdst_v, cout_v, pdst_v,
                    sem_c, sem_p):
    wid = lax.axis_index("s") * NC + lax.axis_index("c")
    pltpu.sync_copy(cust_idx_hbm.at[wid], cidx_v)
    pltpu.sync_copy(prod_idx_hbm.at[wid], pidx_v)
    pgather = pltpu.async_copy(w_prod_hbm.at[pidx_v], pdst_v, sem_p)

    for c in range(B_PER_W // RCHUNK):
        def build(v, carry, c=c):
            idxv = cidx_v[pl.ds(c * RCHUNK + v * 16, 16)]
            q0 = (idxv * CUST_DIM) >> 7
            glist_v[pl.ds(v * 16, 16)] = q0
            glist_v[pl.ds(RCHUNK + v * 16, 16)] = q0 + 1
            return carry

        lax.fori_loop(0, RCHUNK // 16, build, 0)
        pltpu.async_copy(w_cust_hbm.at[glist_v], gdst_v, sem_c).wait()

        def trim(v, carry, c=c):
            idxv = cidx_v[pl.ds(c * RCHUNK + v * 16, 16)]
            for l in range(16):
                r = v * 16 + l
                k = c * RCHUNK + r
                w0 = idxv[l] * CUST_DIM
                a = w0 & 96
                b2 = a + 32
                row2 = r + (b2 >> 7) * RCHUNK
                col2 = b2 & 127
                cout_v[k, pl.ds(0, 16)] = gdst_v[r, pl.ds(a, 16)]
                cout_v[k, pl.ds(16, 16)] = gdst_v[r, pl.ds(a + 16, 16)]
                cout_v[k, pl.ds(32, 16)] = gdst_v[row2, pl.ds(col2, 16)]
                cout_v[k, pl.ds(48, 16)] = gdst_v[row2, pl.ds(col2 + 16, 16)]
            return carry

        lax.fori_loop(0, RCHUNK // 16, trim, 0)

    pgather.wait()
    pltpu.sync_copy(cout_v, cust_out_hbm.at[wid])
    pltpu.sync_copy(pdst_v, prod_out_hbm.at[wid])


@functools.cache
def _sc_gather_kernel():
    mesh = plsc.VectorSubcoreMesh(core_axis_name="c", subcore_axis_name="s")
    return pl.kernel(
        _sc_gather_body,
        out_type=(
            jax.ShapeDtypeStruct((NW, B_PER_W, WIN), jnp.float32),
            jax.ShapeDtypeStruct((NW, B_PER_W, PROD_PAD), jnp.float32),
        ),
        mesh=mesh,
        scratch_types=[
            pltpu.VMEM((B_PER_W,), jnp.int32),
            pltpu.VMEM((B_PER_W,), jnp.int32),
            pltpu.VMEM((2 * RCHUNK,), jnp.int32),
            pltpu.VMEM((2 * RCHUNK, GRAN), jnp.float32),
            pltpu.VMEM((B_PER_W, WIN), jnp.float32),
            pltpu.VMEM((B_PER_W, PROD_PAD), jnp.float32),
            pltpu.SemaphoreType.DMA,
            pltpu.SemaphoreType.DMA,
        ],
        compiler_params=pltpu.CompilerParams(use_tc_tiling_on_sc=False),
    )


def _mlp_body(cust_ref, cidx_ref, prod_ref, dense_ref,
              w1_ref, b1_ref, w2_ref, b2_ref, w3_ref, b3_ref, out_ref):
    x64 = cust_ref[...]
    bm = x64.shape[0]
    shift = (cidx_ref[...] * CUST_DIM) & 31
    x24 = jnp.zeros((bm, CUST_PAD), jnp.float32)
    for s in range(0, 32, 2):
        m = (shift == s).astype(jnp.float32)
        x24 = x24 + m * x64[:, s:s + CUST_PAD]
    x = jnp.concatenate([x24, prod_ref[...], dense_ref[...]], axis=1)
    h = x @ w1_ref[...] + b1_ref[...]
    h = jnp.maximum(h, 0.0)
    h = h @ w2_ref[...] + b2_ref[...]
    h = jnp.maximum(h, 0.0)
    o = h @ w3_ref[...] + b3_ref[...]
    out_ref[...] = jax.nn.sigmoid(o)


def _tc_mlp(cust64, cidx2, prod_emb, dense, W1p, b1, W2, b2, W3, b3,
            block_m=2048):
    grid = (BATCH // block_m,)
    full = lambda shape: pl.BlockSpec(shape, lambda i: (0, 0))
    return pl.pallas_call(
        _mlp_body,
        grid=grid,
        in_specs=[
            pl.BlockSpec((block_m, WIN), lambda i: (i, 0)),
            pl.BlockSpec((block_m, 1), lambda i: (i, 0)),
            pl.BlockSpec((block_m, PROD_PAD), lambda i: (i, 0)),
            pl.BlockSpec((block_m, DENSE_DIM), lambda i: (i, 0)),
            full(W1p.shape),
            full(b1.shape),
            full(W2.shape),
            full(b2.shape),
            full(W3.shape),
            full(b3.shape),
        ],
        out_specs=pl.BlockSpec((block_m, 1), lambda i: (i, 0)),
        out_shape=jax.ShapeDtypeStruct((BATCH, 1), jnp.float32),
        compiler_params=pltpu.CompilerParams(
            dimension_semantics=("arbitrary",)),
    )(cust64, cidx2, prod_emb, dense, W1p, b1, W2, b2, W3, b3)


def kernel(core_cust_id_input, prod_code_input, dense_input, W_cust, W_prod,
           W1, b1, W2, b2, W3, b3):
    cidx = core_cust_id_input.reshape(NW, B_PER_W)
    pidx = prod_code_input.reshape(NW, B_PER_W)
    w_cust_g = jnp.concatenate(
        [W_cust.reshape(-1), jnp.zeros((TAB_PAD_WORDS,), jnp.float32)]
    ).reshape(TAB_GROWS, GRAN)
    w_prod_p = jnp.pad(W_prod, ((0, 0), (0, PROD_PAD - PROD_DIM)))
    cust64, prod_emb = _sc_gather_kernel()(cidx, pidx, w_cust_g, w_prod_p)
    cust64 = cust64.reshape(BATCH, WIN)
    prod_emb = prod_emb.reshape(BATCH, PROD_PAD)
    cidx2 = core_cust_id_input.reshape(BATCH, 1)
    zc = jnp.zeros((CUST_PAD - CUST_DIM, H0), jnp.float32)
    zp = jnp.zeros((PROD_PAD - PROD_DIM, H0), jnp.float32)
    W1p = jnp.concatenate(
        [W1[0:CUST_DIM], zc, W1[CUST_DIM:CUST_DIM + PROD_DIM], zp,
         W1[CUST_DIM + PROD_DIM:]], axis=0)
    return _tc_mlp(
        cust64, cidx2, prod_emb, dense_input,
        W1p, b1.reshape(1, H0), W2, b2.reshape(1, H1), W3, b3.reshape(1, 1))


# speedup vs baseline: 1.0673x; 1.0673x over previous
import functools

import jax
import jax.numpy as jnp
from jax import lax
from jax.experimental import pallas as pl
from jax.experimental.pallas import tpu as pltpu
from jax.experimental.pallas import tpu_sc as plsc

BATCH = 16384
CUST_VOCAB = 264055
CUST_DIM = 18
PROD_DIM = 7
DENSE_DIM = 13
CUST_PAD = 24
PROD_PAD = 8
H0, H1 = 1024, 512

NC, NS = 2, 16
NW = NC * NS
B_PER_W = BATCH // NW

GRAN = 8
GPR = 4
NGR = B_PER_W * GPR
TAB_WORDS = CUST_VOCAB * CUST_DIM
TAB_GROWS = ((CUST_VOCAB - 1) * CUST_DIM) // GRAN + GPR + 1
TAB_PAD_WORDS = TAB_GROWS * GRAN - TAB_WORDS


def _sc_gather_body(cust_idx_hbm, prod_idx_hbm, w_cust_hbm, w_prod_hbm,
                    cust_out_hbm, prod_out_hbm,
                    cidx_v, pidx_v, glist_v, gdst_v, pdst_v, sem):
    wid = lax.axis_index("s") * NC + lax.axis_index("c")
    pltpu.sync_copy(cust_idx_hbm.at[wid], cidx_v)
    pltpu.sync_copy(prod_idx_hbm.at[wid], pidx_v)

    def build(v, carry):
        idxv = cidx_v[pl.ds(v * 16, 16)]
        g0 = (idxv * CUST_DIM) >> 3
        for j in range(GPR):
            glist_v[pl.ds(j * B_PER_W + v * 16, 16)] = g0 + j
        return carry

    lax.fori_loop(0, B_PER_W // 16, build, 0)

    pgather = pltpu.async_copy(w_prod_hbm.at[pidx_v], pdst_v, sem)
    cgather = pltpu.async_copy(w_cust_hbm.at[glist_v], gdst_v, sem)
    pgather.wait()
    cgather.wait()

    pltpu.sync_copy(gdst_v, cust_out_hbm.at[wid])
    pltpu.sync_copy(pdst_v, prod_out_hbm.at[wid])


@functools.cache
def _sc_gather_kernel():
    mesh = plsc.VectorSubcoreMesh(core_axis_name="c", subcore_axis_name="s")
    return pl.kernel(
        _sc_gather_body,
        out_type=(
            jax.ShapeDtypeStruct((NW, NGR, GRAN), jnp.float32),
            jax.ShapeDtypeStruct((NW, B_PER_W, PROD_PAD), jnp.float32),
        ),
        mesh=mesh,
        scratch_types=[
            pltpu.VMEM((B_PER_W,), jnp.int32),
            pltpu.VMEM((B_PER_W,), jnp.int32),
            pltpu.VMEM((NGR,), jnp.int32),
            pltpu.VMEM((NGR, GRAN), jnp.float32),
            pltpu.VMEM((B_PER_W, PROD_PAD), jnp.float32),
            pltpu.SemaphoreType.DMA,
        ],
        compiler_params=pltpu.CompilerParams(use_tc_tiling_on_sc=False),
    )


def _mlp_body(c0_ref, c1_ref, c2_ref, c3_ref, cidx_ref, prod_ref, dense_ref,
              w1_ref, b1_ref, w2_ref, b2_ref, w3_ref, b3_ref, out_ref):
    bm = B_PER_W
    planes = [jnp.reshape(r[...], (bm, GRAN)) for r in
              (c0_ref, c1_ref, c2_ref, c3_ref)]
    x32 = jnp.concatenate(planes, axis=1)
    shift = (cidx_ref[...] * CUST_DIM) & 7
    x24 = jnp.zeros((bm, CUST_PAD), jnp.float32)
    for s in (0, 2, 4, 6):
        m = (shift == s).astype(jnp.float32)
        x24 = x24 + m * x32[:, s:s + CUST_PAD]
    x = jnp.concatenate([x24, prod_ref[...], dense_ref[...]], axis=1)
    h = x @ w1_ref[...] + b1_ref[...]
    h = jnp.maximum(h, 0.0)
    h = jnp.matmul(h.astype(jnp.bfloat16), w2_ref[...],
                   preferred_element_type=jnp.float32) + b2_ref[...]
    h = jnp.maximum(h, 0.0)
    o = h @ w3_ref[...] + b3_ref[...]
    out_ref[...] = jax.nn.sigmoid(o)


def _tc_mlp(cust_raw, cidx2, prod_emb, dense, W1p, b1, W2bf, b2, W3, b3):
    bm = B_PER_W
    grid = (NW,)
    full = lambda shape: pl.BlockSpec(shape, lambda i: (0, 0))

    def plane_spec(j):
        return pl.BlockSpec((1, 1, bm, GRAN), lambda i, j=j: (i, j, 0, 0))

    return pl.pallas_call(
        _mlp_body,
        grid=grid,
        in_specs=[
            plane_spec(0), plane_spec(1), plane_spec(2), plane_spec(3),
            pl.BlockSpec((bm, 1), lambda i: (i, 0)),
            pl.BlockSpec((bm, PROD_PAD), lambda i: (i, 0)),
            pl.BlockSpec((bm, DENSE_DIM), lambda i: (i, 0)),
            full(W1p.shape),
            full(b1.shape),
            full(W2bf.shape),
            full(b2.shape),
            full(W3.shape),
            full(b3.shape),
        ],
        out_specs=pl.BlockSpec((bm, 1), lambda i: (i, 0)),
        out_shape=jax.ShapeDtypeStruct((BATCH, 1), jnp.float32),
        compiler_params=pltpu.CompilerParams(
            dimension_semantics=("arbitrary",)),
    )(cust_raw, cust_raw, cust_raw, cust_raw, cidx2, prod_emb, dense,
      W1p, b1, W2bf, b2, W3, b3)


def kernel(core_cust_id_input, prod_code_input, dense_input, W_cust, W_prod,
           W1, b1, W2, b2, W3, b3):
    cidx = core_cust_id_input.reshape(NW, B_PER_W)
    pidx = prod_code_input.reshape(NW, B_PER_W)
    w_cust_g = jnp.concatenate(
        [W_cust.reshape(-1), jnp.zeros((TAB_PAD_WORDS,), jnp.float32)]
    ).reshape(TAB_GROWS, GRAN)
    w_prod_p = jnp.pad(W_prod, ((0, 0), (0, PROD_PAD - PROD_DIM)))
    cust_raw, prod_emb = _sc_gather_kernel()(cidx, pidx, w_cust_g, w_prod_p)
    cust_raw = cust_raw.reshape(NW, GPR, B_PER_W, GRAN)
    prod_emb = prod_emb.reshape(BATCH, PROD_PAD)
    cidx2 = core_cust_id_input.reshape(BATCH, 1)
    zc = jnp.zeros((CUST_PAD - CUST_DIM, H0), jnp.float32)
    zp = jnp.zeros((PROD_PAD - PROD_DIM, H0), jnp.float32)
    W1p = jnp.concatenate(
        [W1[0:CUST_DIM], zc, W1[CUST_DIM:CUST_DIM + PROD_DIM], zp,
         W1[CUST_DIM + PROD_DIM:]], axis=0)
    return _tc_mlp(
        cust_raw, cidx2, prod_emb, dense_input,
        W1p, b1.reshape(1, H0), W2.astype(jnp.bfloat16), b2.reshape(1, H1),
        W3, b3.reshape(1, 1))
